# baseline (speedup 1.0000x reference)
import functools

import jax
import jax.numpy as jnp
from jax.experimental import pallas as pl
from jax.experimental.pallas import tpu as pltpu

N_BUCKETS = 512
S = 32
TAU = 0.1
ALPHA = 0.5
TB = 32


def _token_kernel(g_per_row, nsteps,
                  rows_ref, chg_ref, slots_ref,
                  q_ref,
                  tid_ref,
                  bk_ref,
                  keys_hbm,
                  vals_hbm,
                  cb_ref,
                  stid_ref,
                  out_ref,
                  sim_ref,
                  k_buf, v_buf,
                  ksem, vsem):
    i = pl.program_id(0)
    b = i // g_per_row
    stride = nsteps + 1

    def row_of(j, step):
        return rows_ref[j * stride + step]

    def start_kv(j, row, slot):
        bb = row // N_BUCKETS
        off = (row % N_BUCKETS) * S
        pltpu.make_async_copy(
            keys_hbm.at[bb, pl.ds(off, S), :],
            k_buf.at[slot, pl.ds(j * S, S), :], ksem.at[slot, j]).start()
        pltpu.make_async_copy(
            vals_hbm.at[bb, pl.ds(off, S), :],
            v_buf.at[slot, pl.ds(j * S, S), :], vsem.at[slot, j]).start()

    def wait_kv(j, row, slot):
        bb = row // N_BUCKETS
        off = (row % N_BUCKETS) * S
        pltpu.make_async_copy(
            keys_hbm.at[bb, pl.ds(off, S), :],
            k_buf.at[slot, pl.ds(j * S, S), :], ksem.at[slot, j]).wait()
        pltpu.make_async_copy(
            vals_hbm.at[bb, pl.ds(off, S), :],
            v_buf.at[slot, pl.ds(j * S, S), :], vsem.at[slot, j]).wait()

    @pl.when(i == 0)
    def _():
        for j in range(TB):
            start_kv(j, row_of(j, 0), 0)
        for j in range(TB):
            wait_kv(j, row_of(j, 0), 0)

    @pl.when(i > 0)
    def _():
        for j in range(TB):
            @pl.when(chg_ref[j * stride + i] > 0)
            def _(j=j):
                wait_kv(j, row_of(j, i), slots_ref[j * stride + i])

    for j in range(TB):
        @pl.when(chg_ref[j * stride + i + 1] > 0)
        def _(j=j):
            start_kv(j, row_of(j, i + 1), slots_ref[j * stride + i + 1])

    qs = q_ref[0, 0]
    qn = qs * jax.lax.rsqrt(
        jnp.maximum(jnp.sum(qs * qs, axis=1, keepdims=True), 1e-24))

    bk_col = bk_ref[0, 0]
    iota = jax.lax.broadcasted_iota(jnp.int32, (TB, N_BUCKETS), 1)
    onehot = (iota == bk_col).astype(jnp.float32)
    anchors = jax.lax.dot_general(
        onehot, cb_ref[...], (((1,), (0,)), ((), ())),
        preferred_element_type=jnp.float32)

    uq = ALPHA * qn + (1.0 - ALPHA) * anchors
    uq = uq * jax.lax.rsqrt(
        jnp.maximum(jnp.sum(uq * uq, axis=1, keepdims=True), 1e-24))

    scores = jnp.concatenate([
        jax.lax.dot_general(
            uq[j:j + 1, :], k_buf[slots_ref[j * stride + i], pl.ds(j * S, S), :],
            (((1,), (1,)), ((), ())),
            preferred_element_type=jnp.float32)
        for j in range(TB)], axis=0)

    stid_b = stid_ref[pl.ds(b * N_BUCKETS, N_BUCKETS), :]
    stids = jax.lax.dot_general(
        onehot, stid_b, (((1,), (0,)), ((), ())),
        preferred_element_type=jnp.float32)
    tid_col = tid_ref[0, 0].astype(jnp.float32)
    mask = (stids == tid_col).astype(jnp.float32)
    msum = jnp.sum(mask, axis=1, keepdims=True)
    has_match = msum > 0.0

    probs_hard = mask / (msum + 1e-9)
    s2 = scores * (1.0 / TAU)
    smax = jnp.max(s2, axis=1, keepdims=True)
    e = jnp.exp(s2 - smax)
    probs_soft = e / jnp.sum(e, axis=1, keepdims=True)
    probs = jnp.where(has_match, probs_hard, probs_soft)

    vals = jnp.concatenate([
        jax.lax.dot_general(
            probs[j:j + 1, :], v_buf[slots_ref[j * stride + i], pl.ds(j * S, S), :],
            (((1,), (0,)), ((), ())),
            preferred_element_type=jnp.float32)
        for j in range(TB)], axis=0)
    out_ref[0, 0] = vals

    max_scores = jnp.max(scores, axis=1, keepdims=True)
    sim = jnp.where(has_match, 10.0, max_scores)
    sim_ref[0, 0] = sim * jnp.ones((1, 128), jnp.float32)


@jax.jit
def kernel(query_emb, slot_values, slot_keys, tids, centroid_codebook,
           slot_tids):
    B, T, D = query_emb.shape
    G = T // TB
    nsteps = B * G
    buckets = tids % N_BUCKETS

    order = jnp.argsort(buckets, axis=-1)
    pi = order.reshape(B, TB, G).transpose(0, 2, 1).reshape(B, T)
    q_cm = jnp.take_along_axis(query_emb, pi[:, :, None], axis=1)
    tid_cm = jnp.take_along_axis(tids, pi, axis=1)
    bk_cm = jnp.take_along_axis(buckets, pi, axis=1).reshape(B * T)

    sb = jnp.take_along_axis(buckets, order, axis=1)
    rows_bt = (sb + jnp.arange(B, dtype=sb.dtype)[:, None] * N_BUCKETS)
    rows_ji = rows_bt.reshape(B, TB, G).transpose(1, 0, 2).reshape(
        TB, nsteps)
    rows_tab = jnp.concatenate(
        [rows_ji, rows_ji[:, -1:]], axis=1).reshape(TB * (nsteps + 1))
    chg_ji = (rows_ji[:, 1:] != rows_ji[:, :-1]).astype(jnp.int32)
    zero_col = jnp.zeros((TB, 1), jnp.int32)
    chg_pad = jnp.concatenate([zero_col, chg_ji, zero_col], axis=1)
    slots_pad = jnp.cumsum(chg_pad, axis=1) % 2
    chg_tab = chg_pad.reshape(TB * (nsteps + 1))
    slots_tab = slots_pad.reshape(TB * (nsteps + 1))

    stid_tab = slot_tids.reshape(B * N_BUCKETS, S).astype(jnp.float32)
    q4 = q_cm.reshape(B, G, TB, D)
    tid4 = tid_cm.reshape(B, G, TB, 1)
    bk4 = bk_cm.reshape(B, G, TB, 1)

    def q_map(i, rows, chg, slots):
        return (i // G, i % G, 0, 0)

    def cb_map(i, rows, chg, slots):
        return (0, 0)

    in_specs = [pl.BlockSpec((1, 1, TB, D), q_map),
                pl.BlockSpec((1, 1, TB, 1), q_map),
                pl.BlockSpec((1, 1, TB, 1), q_map),
                pl.BlockSpec(memory_space=pl.ANY),
                pl.BlockSpec(memory_space=pl.ANY),
                pl.BlockSpec((N_BUCKETS, D), cb_map),
                pl.BlockSpec((B * N_BUCKETS, S), cb_map)]

    grid_spec = pltpu.PrefetchScalarGridSpec(
        num_scalar_prefetch=3,
        grid=(nsteps,),
        in_specs=in_specs,
        out_specs=[
            pl.BlockSpec((1, 1, TB, D), q_map),
            pl.BlockSpec((1, 1, TB, 128), q_map),
        ],
        scratch_shapes=[
            pltpu.VMEM((2, TB * S, D), jnp.float32),
            pltpu.VMEM((2, TB * S, D), jnp.float32),
            pltpu.SemaphoreType.DMA((2, TB)),
            pltpu.SemaphoreType.DMA((2, TB)),
        ],
    )

    out_cm, sim_cm = pl.pallas_call(
        functools.partial(_token_kernel, G, nsteps),
        grid_spec=grid_spec,
        out_shape=[
            jax.ShapeDtypeStruct((B, G, TB, D), jnp.float32),
            jax.ShapeDtypeStruct((B, G, TB, 128), jnp.float32),
        ],
    )(rows_tab, chg_tab, slots_tab, q4, tid4, bk4, slot_keys,
      slot_values, centroid_codebook, stid_tab)

    inv = jnp.argsort(order, axis=-1)
    phi = (inv % G) * TB + (inv // G)
    out = jnp.take_along_axis(
        out_cm.reshape(B, T, D), phi[:, :, None], axis=1)
    sim = jnp.take_along_axis(
        sim_cm[:, :, :, 0].reshape(B, T), phi, axis=1)
    return out, sim


# speedup vs baseline: 1.0264x; 1.0264x over previous
import functools

import jax
import jax.numpy as jnp
from jax.experimental import pallas as pl
from jax.experimental.pallas import tpu as pltpu

N_BUCKETS = 512
S = 32
TAU = 0.1
ALPHA = 0.5
TB = 32


def _token_kernel(g_per_row, nsteps,
                  bk_cm_ref, rows_ref, chg_ref, slots_ref,
                  q_ref,
                  tid_ref,
                  keys_hbm,
                  vals_hbm,
                  cb_ref,
                  stid_ref,
                  out_ref,
                  sim_ref,
                  k_buf, v_buf,
                  ksem, vsem):
    i = pl.program_id(0)
    base = i * TB
    b = i // g_per_row
    stride = nsteps + 1

    def row_of(j, step):
        return rows_ref[j * stride + step]

    def start_kv(j, row, slot):
        bb = row // N_BUCKETS
        off = (row % N_BUCKETS) * S
        pltpu.make_async_copy(
            keys_hbm.at[bb, pl.ds(off, S), :],
            k_buf.at[slot, pl.ds(j * S, S), :], ksem.at[slot, j]).start()
        pltpu.make_async_copy(
            vals_hbm.at[bb, pl.ds(off, S), :],
            v_buf.at[slot, pl.ds(j * S, S), :], vsem.at[slot, j]).start()

    def wait_kv(j, row, slot):
        bb = row // N_BUCKETS
        off = (row % N_BUCKETS) * S
        pltpu.make_async_copy(
            keys_hbm.at[bb, pl.ds(off, S), :],
            k_buf.at[slot, pl.ds(j * S, S), :], ksem.at[slot, j]).wait()
        pltpu.make_async_copy(
            vals_hbm.at[bb, pl.ds(off, S), :],
            v_buf.at[slot, pl.ds(j * S, S), :], vsem.at[slot, j]).wait()

    @pl.when(i == 0)
    def _():
        for j in range(TB):
            start_kv(j, row_of(j, 0), 0)
        for j in range(TB):
            wait_kv(j, row_of(j, 0), 0)

    @pl.when(i > 0)
    def _():
        for j in range(TB):
            @pl.when(chg_ref[j * stride + i] > 0)
            def _(j=j):
                wait_kv(j, row_of(j, i), slots_ref[j * stride + i])

    for j in range(TB):
        @pl.when(chg_ref[j * stride + i + 1] > 0)
        def _(j=j):
            start_kv(j, row_of(j, i + 1), slots_ref[j * stride + i + 1])

    qs = q_ref[0, 0]
    qn = qs * jax.lax.rsqrt(
        jnp.maximum(jnp.sum(qs * qs, axis=1, keepdims=True), 1e-24))

    anchors = jnp.concatenate(
        [cb_ref[pl.ds(bk_cm_ref[base + j], 1), :] for j in range(TB)],
        axis=0)

    uq = ALPHA * qn + (1.0 - ALPHA) * anchors
    uq = uq * jax.lax.rsqrt(
        jnp.maximum(jnp.sum(uq * uq, axis=1, keepdims=True), 1e-24))

    scores = jnp.concatenate([
        jax.lax.dot_general(
            uq[j:j + 1, :], k_buf[slots_ref[j * stride + i], pl.ds(j * S, S), :],
            (((1,), (1,)), ((), ())),
            preferred_element_type=jnp.float32)
        for j in range(TB)], axis=0)

    row0 = b * N_BUCKETS
    stids = jnp.concatenate(
        [stid_ref[pl.ds(row0 + bk_cm_ref[base + j], 1), :]
         for j in range(TB)], axis=0)
    tid_col = tid_ref[0, 0]
    mask = (stids == tid_col).astype(jnp.float32)
    msum = jnp.sum(mask, axis=1, keepdims=True)
    has_match = msum > 0.0

    probs_hard = mask / (msum + 1e-9)
    s2 = scores * (1.0 / TAU)
    smax = jnp.max(s2, axis=1, keepdims=True)
    e = jnp.exp(s2 - smax)
    probs_soft = e / jnp.sum(e, axis=1, keepdims=True)
    probs = jnp.where(has_match, probs_hard, probs_soft)

    vals = jnp.concatenate([
        jax.lax.dot_general(
            probs[j:j + 1, :], v_buf[slots_ref[j * stride + i], pl.ds(j * S, S), :],
            (((1,), (0,)), ((), ())),
            preferred_element_type=jnp.float32)
        for j in range(TB)], axis=0)
    out_ref[0, 0] = vals

    max_scores = jnp.max(scores, axis=1, keepdims=True)
    sim = jnp.where(has_match, 10.0, max_scores)
    sim_ref[0, 0] = sim * jnp.ones((1, 128), jnp.float32)


@jax.jit
def kernel(query_emb, slot_values, slot_keys, tids, centroid_codebook,
           slot_tids):
    B, T, D = query_emb.shape
    G = T // TB
    nsteps = B * G
    buckets = tids % N_BUCKETS

    order = jnp.argsort(buckets, axis=-1)
    pi = order.reshape(B, TB, G).transpose(0, 2, 1).reshape(B, T)
    q_cm = jnp.take_along_axis(query_emb, pi[:, :, None], axis=1)
    tid_cm = jnp.take_along_axis(tids, pi, axis=1)
    bk_cm = jnp.take_along_axis(buckets, pi, axis=1).reshape(B * T)

    sb = jnp.take_along_axis(buckets, order, axis=1)
    rows_bt = (sb + jnp.arange(B, dtype=sb.dtype)[:, None] * N_BUCKETS)
    rows_ji = rows_bt.reshape(B, TB, G).transpose(1, 0, 2).reshape(
        TB, nsteps)
    rows_tab = jnp.concatenate(
        [rows_ji, rows_ji[:, -1:]], axis=1).reshape(TB * (nsteps + 1))
    chg_ji = (rows_ji[:, 1:] != rows_ji[:, :-1]).astype(jnp.int32)
    zero_col = jnp.zeros((TB, 1), jnp.int32)
    chg_pad = jnp.concatenate([zero_col, chg_ji, zero_col], axis=1)
    slots_pad = jnp.cumsum(chg_pad, axis=1) % 2
    chg_tab = chg_pad.reshape(TB * (nsteps + 1))
    slots_tab = slots_pad.reshape(TB * (nsteps + 1))

    stid_tab = slot_tids.reshape(B * N_BUCKETS, S)
    q4 = q_cm.reshape(B, G, TB, D)
    tid4 = tid_cm.reshape(B, G, TB, 1)

    def q_map(i, bkcm, rows, chg, slots):
        return (i // G, i % G, 0, 0)

    def cb_map(i, bkcm, rows, chg, slots):
        return (0, 0)

    in_specs = [pl.BlockSpec((1, 1, TB, D), q_map),
                pl.BlockSpec((1, 1, TB, 1), q_map),
                pl.BlockSpec(memory_space=pl.ANY),
                pl.BlockSpec(memory_space=pl.ANY),
                pl.BlockSpec((N_BUCKETS, D), cb_map),
                pl.BlockSpec((B * N_BUCKETS, S), cb_map)]

    grid_spec = pltpu.PrefetchScalarGridSpec(
        num_scalar_prefetch=4,
        grid=(nsteps,),
        in_specs=in_specs,
        out_specs=[
            pl.BlockSpec((1, 1, TB, D), q_map),
            pl.BlockSpec((1, 1, TB, 128), q_map),
        ],
        scratch_shapes=[
            pltpu.VMEM((2, TB * S, D), jnp.float32),
            pltpu.VMEM((2, TB * S, D), jnp.float32),
            pltpu.SemaphoreType.DMA((2, TB)),
            pltpu.SemaphoreType.DMA((2, TB)),
        ],
    )

    out_cm, sim_cm = pl.pallas_call(
        functools.partial(_token_kernel, G, nsteps),
        grid_spec=grid_spec,
        out_shape=[
            jax.ShapeDtypeStruct((B, G, TB, D), jnp.float32),
            jax.ShapeDtypeStruct((B, G, TB, 128), jnp.float32),
        ],
    )(bk_cm, rows_tab, chg_tab, slots_tab, q4, tid4, slot_keys,
      slot_values, centroid_codebook, stid_tab)

    inv = jnp.argsort(order, axis=-1)
    phi = (inv % G) * TB + (inv // G)
    out = jnp.take_along_axis(
        out_cm.reshape(B, T, D), phi[:, :, None], axis=1)
    sim = jnp.take_along_axis(
        sim_cm[:, :, :, 0].reshape(B, T), phi, axis=1)
    return out, sim


# speedup vs baseline: 1.2257x; 1.1942x over previous
import functools

import jax
import jax.numpy as jnp
from jax.experimental import pallas as pl
from jax.experimental.pallas import tpu as pltpu

N_BUCKETS = 512
S = 32
TAU = 0.1
ALPHA = 0.5
TB = 64


def _token_kernel(g_per_row,
                  buckets_ref, tids_pref,
                  q_ref,
                  tid_ref,
                  *refs):
    k_refs = refs[0:TB]
    v_refs = refs[TB:2 * TB]
    cb_ref = refs[2 * TB]
    stid_ref = refs[2 * TB + 1]
    out_ref = refs[2 * TB + 2]
    sim_ref = refs[2 * TB + 3]

    i = pl.program_id(0)
    base = i * TB
    b = i // g_per_row

    qs = q_ref[0, 0]
    qn = qs * jax.lax.rsqrt(
        jnp.maximum(jnp.sum(qs * qs, axis=1, keepdims=True), 1e-24))

    anchors = jnp.concatenate(
        [cb_ref[pl.ds(buckets_ref[base + j], 1), :] for j in range(TB)],
        axis=0)

    uq = ALPHA * qn + (1.0 - ALPHA) * anchors
    uq = uq * jax.lax.rsqrt(
        jnp.maximum(jnp.sum(uq * uq, axis=1, keepdims=True), 1e-24))

    scores = jnp.concatenate([
        jax.lax.dot_general(
            uq[j:j + 1, :], k_refs[j][0], (((1,), (1,)), ((), ())),
            preferred_element_type=jnp.float32)
        for j in range(TB)], axis=0)

    row0 = b * N_BUCKETS
    stids = jnp.concatenate(
        [stid_ref[pl.ds(row0 + buckets_ref[base + j], 1), :]
         for j in range(TB)], axis=0)
    tid_col = tid_ref[0, 0]
    mask = (stids == tid_col).astype(jnp.float32)
    msum = jnp.sum(mask, axis=1, keepdims=True)
    has_match = msum > 0.0

    probs_hard = mask / (msum + 1e-9)
    s2 = scores * (1.0 / TAU)
    smax = jnp.max(s2, axis=1, keepdims=True)
    e = jnp.exp(s2 - smax)
    probs_soft = e / jnp.sum(e, axis=1, keepdims=True)
    probs = jnp.where(has_match, probs_hard, probs_soft)

    vals = jnp.concatenate([
        jax.lax.dot_general(
            probs[j:j + 1, :], v_refs[j][0], (((1,), (0,)), ((), ())),
            preferred_element_type=jnp.float32)
        for j in range(TB)], axis=0)
    out_ref[0, 0] = vals

    max_scores = jnp.max(scores, axis=1, keepdims=True)
    sim = jnp.where(has_match, 10.0, max_scores)
    sim_ref[0, 0] = sim * jnp.ones((1, 128), jnp.float32)


@jax.jit
def kernel(query_emb, slot_values, slot_keys, tids, centroid_codebook,
           slot_tids):
    B, T, D = query_emb.shape
    G = T // TB
    buckets = (tids % N_BUCKETS).reshape(B * T)
    tids_flat = tids.reshape(B * T)
    stid_tab = slot_tids.reshape(B * N_BUCKETS, S)
    q4 = query_emb.reshape(B, G, TB, D)
    tid4 = tids.reshape(B, G, TB, 1)

    grid = (B * G,)

    def q_map(i, bk, tf):
        return (i // G, i % G, 0, 0)

    def kv_map(j):
        def m(i, bk, tf):
            return (i // G, bk[i * TB + j], 0)
        return m

    def cb_map(i, bk, tf):
        return (0, 0)

    in_specs = [pl.BlockSpec((1, 1, TB, D), q_map),
                pl.BlockSpec((1, 1, TB, 1), q_map)]
    in_specs += [pl.BlockSpec((1, S, D), kv_map(j)) for j in range(TB)]
    in_specs += [pl.BlockSpec((1, S, D), kv_map(j)) for j in range(TB)]
    in_specs += [pl.BlockSpec((N_BUCKETS, D), cb_map),
                 pl.BlockSpec((B * N_BUCKETS, S), cb_map)]

    grid_spec = pltpu.PrefetchScalarGridSpec(
        num_scalar_prefetch=2,
        grid=grid,
        in_specs=in_specs,
        out_specs=[
            pl.BlockSpec((1, 1, TB, D), q_map),
            pl.BlockSpec((1, 1, TB, 128), q_map),
        ],
    )

    args = ([buckets, tids_flat, q4, tid4]
            + [slot_keys] * TB + [slot_values] * TB
            + [centroid_codebook, stid_tab])
    out, sim = pl.pallas_call(
        functools.partial(_token_kernel, G),
        grid_spec=grid_spec,
        out_shape=[
            jax.ShapeDtypeStruct((B, G, TB, D), jnp.float32),
            jax.ShapeDtypeStruct((B, G, TB, 128), jnp.float32),
        ],
    )(*args)
    return out.reshape(B, T, D), sim[:, :, :, 0].reshape(B, T)
